# baseline (speedup 1.0000x reference)
import jax
import jax.numpy as jnp
from jax import lax
from jax.experimental import pallas as pl
from jax.experimental.pallas import tpu as pltpu
from jax.experimental.pallas import tpu_sc as plsc

_P = 0.2
_B = 100000
_F = 128
_NC = 2
_NS = 16
_NW = _NC * _NS
_RCH = 160
_NCHW = 7
_BSC = _NW * _NCHW * _RCH
_TROWS = 160


def _sc_body(x_hbm, k_hbm, ti_hbm, tw_hbm, out_hbm,
             xb0, xb1, kb0, kb1, cb0, cb1, tiv, twv,
             semi0, semi1, semo0, semo1):
    wid = lax.axis_index("s") * _NC + lax.axis_index("c")

    pltpu.sync_copy(ti_hbm, tiv)
    pltpu.sync_copy(tw_hbm, twv)

    zero = jnp.float32(0.0)
    scale_keep = jnp.float32(1.0 / (1.0 - _P))

    xbufs = (xb0, xb1)
    kbufs = (kb0, kb1)
    cbufs = (cb0, cb1)
    isems = (semi0, semi1)
    osems = (semo0, semo1)

    def chunk_row(t):
        c = t * _NW + wid
        return pl.multiple_of(c * _RCH, 32)

    def issue_in(t, s):
        r = chunk_row(t)
        hx = pltpu.async_copy(x_hbm.at[pl.ds(r, _RCH), :], xbufs[s], isems[s])
        hk = pltpu.async_copy(k_hbm.at[pl.ds(r, _RCH), :], kbufs[s], isems[s])
        return (hx, hk)

    def issue_out(t, s):
        r = chunk_row(t)
        return pltpu.async_copy(cbufs[s], out_hbm.at[pl.ds(r, _RCH), :],
                                osems[s])

    def compute(s):
        xb = xbufs[s]
        kb = kbufs[s]
        cb = cbufs[s]

        def group_b(g, _):
            off = pl.multiple_of(g * 16, 16)
            i0g = tiv[pl.ds(off, 16)]
            i1g = tiv[pl.ds(off + _F, 16)]
            w0g = twv[pl.ds(off, 16)]
            w1g = twv[pl.ds(off + _F, 16)]

            def row_b(r, _):
                rx = jnp.full((16,), 0, jnp.int32) + r
                g0 = plsc.load_gather(xb, [rx, i0g])
                g1 = plsc.load_gather(xb, [rx, i1g])
                wt = g0 * w0g + g1 * w1g
                kf = kb[r, pl.ds(off, 16)]
                xv = xb[r, pl.ds(off, 16)]
                cb[r, pl.ds(off, 16)] = jnp.where(
                    kf != zero, xv * scale_keep, wt)
                return 0

            lax.fori_loop(0, _RCH, row_b, 0, unroll=4)
            return 0

        lax.fori_loop(0, 8, group_b, 0, unroll=False)

    in_h = [None, None]
    out_h = [None, None]
    in_h[0] = issue_in(0, 0)
    for t in range(_NCHW):
        s = t % 2
        if t + 1 < _NCHW:
            in_h[1 - s] = issue_in(t + 1, 1 - s)
        hx, hk = in_h[s]
        hx.wait()
        hk.wait()
        if out_h[s] is not None:
            out_h[s].wait()
        compute(s)
        out_h[s] = issue_out(t, s)
    out_h[0].wait()
    out_h[1].wait()


def _tc_tail_body(x_ref, k_ref, st_ref, o_ref):
    wt = jnp.dot(x_ref[...], st_ref[...],
                 precision=lax.Precision.HIGHEST,
                 preferred_element_type=jnp.float32)
    o_ref[...] = jnp.where(k_ref[...],
                           x_ref[...] * jnp.float32(1.0 / (1.0 - _P)),
                           wt * jnp.float32(1.0 / (_P + 1e-12)))


def _tc_tail(x, keep, st):
    base = _BSC // _TROWS
    return pl.pallas_call(
        _tc_tail_body,
        out_shape=jax.ShapeDtypeStruct((_B - _BSC, _F), jnp.float32),
        grid=((_B - _BSC) // _TROWS,),
        in_specs=[pl.BlockSpec((_TROWS, _F), lambda i: (base + i, 0)),
                  pl.BlockSpec((_TROWS, _F), lambda i: (base + i, 0)),
                  pl.BlockSpec((_F, _F), lambda i: (0, 0))],
        out_specs=pl.BlockSpec((_TROWS, _F), lambda i: (i, 0)),
    )(x, keep, st)


def kernel(x, m_w, m_idx, keep):
    kf32 = keep[:_BSC].astype(jnp.float32)

    midx = m_idx.astype(jnp.int32)
    inv_p = jnp.float32(1.0 / (_P + 1e-12))
    ti = jnp.concatenate([midx[:, 0], midx[:, 1]])
    tw = jnp.concatenate([m_w[:, 0] * inv_p, m_w[:, 1] * inv_p])

    rows = jnp.arange(_F, dtype=jnp.int32)[:, None]
    st = (jnp.where(midx[None, :, 0] == rows, m_w[None, :, 0], 0.0)
          + jnp.where(midx[None, :, 1] == rows, m_w[None, :, 1], 0.0))

    mesh = plsc.VectorSubcoreMesh(core_axis_name="c", subcore_axis_name="s")
    out = pl.kernel(
        _sc_body,
        out_type=jax.ShapeDtypeStruct((_B, _F), jnp.float32),
        mesh=mesh,
        compiler_params=pltpu.CompilerParams(needs_layout_passes=False),
        scratch_types=[
            pltpu.VMEM((_RCH, _F), jnp.float32),
            pltpu.VMEM((_RCH, _F), jnp.float32),
            pltpu.VMEM((_RCH, _F), jnp.float32),
            pltpu.VMEM((_RCH, _F), jnp.float32),
            pltpu.VMEM((_RCH, _F), jnp.float32),
            pltpu.VMEM((_RCH, _F), jnp.float32),
            pltpu.VMEM((_F * 2,), jnp.int32),
            pltpu.VMEM((_F * 2,), jnp.float32),
            pltpu.SemaphoreType.DMA,
            pltpu.SemaphoreType.DMA,
            pltpu.SemaphoreType.DMA,
            pltpu.SemaphoreType.DMA,
        ],
    )(x, kf32, ti, tw)
    tail = _tc_tail(x, keep, st)
    return lax.dynamic_update_slice(out, tail, (_BSC, 0))


# speedup vs baseline: 2.1290x; 2.1290x over previous
import jax
import jax.numpy as jnp
from jax import lax
from jax.experimental import pallas as pl
from jax.experimental.pallas import tpu as pltpu
from jax.experimental.pallas import tpu_sc as plsc

_P = 0.2
_B = 100000
_F = 128
_NC = 2
_NS = 16
_NW = _NC * _NS
_RCH = 120
_NCHW = 10
_BSC = _NW * _NCHW * _RCH
_TROWS = 800


def _sc_body(x_hbm, k_hbm, ti_hbm, tw_hbm, out_hbm,
             xb0, xb1, kb0, kb1, cb0, cb1, tiv, twv,
             semi0, semi1, semo0, semo1):
    wid = lax.axis_index("s") * _NC + lax.axis_index("c")

    pltpu.sync_copy(ti_hbm, tiv)
    pltpu.sync_copy(tw_hbm, twv)

    zero = jnp.float32(0.0)
    scale_keep = jnp.float32(1.0 / (1.0 - _P))

    xbufs = (xb0, xb1)
    kbufs = (kb0, kb1)
    cbufs = (cb0, cb1)
    isems = (semi0, semi1)
    osems = (semo0, semo1)

    def chunk_row(t):
        c = t * _NW + wid
        return pl.multiple_of(c * _RCH, 8)

    def issue_in(t, s):
        r = chunk_row(t)
        hx = pltpu.async_copy(x_hbm.at[pl.ds(r, _RCH), :], xbufs[s], isems[s])
        hk = pltpu.async_copy(k_hbm.at[pl.ds(r, _RCH), :], kbufs[s], isems[s])
        return (hx, hk)

    def issue_out(t, s):
        r = chunk_row(t)
        return pltpu.async_copy(cbufs[s], out_hbm.at[pl.ds(r, _RCH), :],
                                osems[s])

    def compute(s):
        xb = xbufs[s]
        kb = kbufs[s]
        cb = cbufs[s]

        def group_b(g, _):
            off = pl.multiple_of(g * 16, 16)
            i0g = tiv[pl.ds(off, 16)]
            i1g = tiv[pl.ds(off + _F, 16)]
            w0g = twv[pl.ds(off, 16)]
            w1g = twv[pl.ds(off + _F, 16)]

            def row_b(r, _):
                rx = jnp.full((16,), 0, jnp.int32) + r
                g0 = plsc.load_gather(xb, [rx, i0g])
                g1 = plsc.load_gather(xb, [rx, i1g])
                wt = g0 * w0g + g1 * w1g
                kf = kb[r, pl.ds(off, 16)]
                xv = xb[r, pl.ds(off, 16)]
                cb[r, pl.ds(off, 16)] = jnp.where(
                    kf != zero, xv * scale_keep, wt)
                return 0

            lax.fori_loop(0, _RCH, row_b, 0, unroll=4)
            return 0

        lax.fori_loop(0, 8, group_b, 0, unroll=False)

    in_h = [None, None]
    out_h = [None, None]
    in_h[0] = issue_in(0, 0)
    for t in range(_NCHW):
        s = t % 2
        if t + 1 < _NCHW:
            in_h[1 - s] = issue_in(t + 1, 1 - s)
        hx, hk = in_h[s]
        hx.wait()
        hk.wait()
        if out_h[s] is not None:
            out_h[s].wait()
        compute(s)
        out_h[s] = issue_out(t, s)
    out_h[0].wait()
    out_h[1].wait()


def _tc_tail_body(x_ref, k_ref, st_ref, o_ref):
    wt = jnp.dot(x_ref[...], st_ref[...],
                 precision=lax.Precision.HIGHEST,
                 preferred_element_type=jnp.float32)
    o_ref[...] = jnp.where(k_ref[...],
                           x_ref[...] * jnp.float32(1.0 / (1.0 - _P)),
                           wt * jnp.float32(1.0 / (_P + 1e-12)))


def _tc_tail(x, keep, st):
    base = _BSC // _TROWS
    return pl.pallas_call(
        _tc_tail_body,
        out_shape=jax.ShapeDtypeStruct((_B - _BSC, _F), jnp.float32),
        grid=((_B - _BSC) // _TROWS,),
        in_specs=[pl.BlockSpec((_TROWS, _F), lambda i: (base + i, 0)),
                  pl.BlockSpec((_TROWS, _F), lambda i: (base + i, 0)),
                  pl.BlockSpec((_F, _F), lambda i: (0, 0))],
        out_specs=pl.BlockSpec((_TROWS, _F), lambda i: (i, 0)),
    )(x, keep, st)


def kernel(x, m_w, m_idx, keep):
    kf32 = keep[:_BSC].astype(jnp.float32)

    midx = m_idx.astype(jnp.int32)
    inv_p = jnp.float32(1.0 / (_P + 1e-12))
    ti = jnp.concatenate([midx[:, 0], midx[:, 1]])
    tw = jnp.concatenate([m_w[:, 0] * inv_p, m_w[:, 1] * inv_p])

    rows = jnp.arange(_F, dtype=jnp.int32)[:, None]
    st = (jnp.where(midx[None, :, 0] == rows, m_w[None, :, 0], 0.0)
          + jnp.where(midx[None, :, 1] == rows, m_w[None, :, 1], 0.0))

    mesh = plsc.VectorSubcoreMesh(core_axis_name="c", subcore_axis_name="s")
    out = pl.kernel(
        _sc_body,
        out_type=jax.ShapeDtypeStruct((_B, _F), jnp.float32),
        mesh=mesh,
        compiler_params=pltpu.CompilerParams(needs_layout_passes=False),
        scratch_types=[
            pltpu.VMEM((_RCH, _F), jnp.float32),
            pltpu.VMEM((_RCH, _F), jnp.float32),
            pltpu.VMEM((_RCH, _F), jnp.float32),
            pltpu.VMEM((_RCH, _F), jnp.float32),
            pltpu.VMEM((_RCH, _F), jnp.float32),
            pltpu.VMEM((_RCH, _F), jnp.float32),
            pltpu.VMEM((_F * 2,), jnp.int32),
            pltpu.VMEM((_F * 2,), jnp.float32),
            pltpu.SemaphoreType.DMA,
            pltpu.SemaphoreType.DMA,
            pltpu.SemaphoreType.DMA,
            pltpu.SemaphoreType.DMA,
        ],
    )(x, kf32, ti, tw)
    tail = _tc_tail(x, keep, st)
    return lax.dynamic_update_slice(out, tail, (_BSC, 0))


# speedup vs baseline: 2.2411x; 1.0527x over previous
import jax
import jax.numpy as jnp
from jax import lax
from jax.experimental import pallas as pl
from jax.experimental.pallas import tpu as pltpu
from jax.experimental.pallas import tpu_sc as plsc

_P = 0.2
_B = 100000
_F = 128
_NC = 2
_NS = 16
_NW = _NC * _NS
_RCH = 120
_NCHW = 10
_BSC = _NW * _NCHW * _RCH
_TROWS = 800


def _sc_body(x_hbm, k_hbm, ti_hbm, tw_hbm, out_hbm,
             xb0, xb1, kb0, kb1, cb0, cb1, tiv, twv,
             semi0, semi1, semo0, semo1):
    wid = lax.axis_index("s") * _NC + lax.axis_index("c")

    pltpu.sync_copy(ti_hbm, tiv)
    pltpu.sync_copy(tw_hbm, twv)

    zero = jnp.float32(0.0)
    scale_keep = jnp.float32(1.0 / (1.0 - _P))

    xbufs = (xb0, xb1)
    kbufs = (kb0, kb1)
    cbufs = (cb0, cb1)
    isems = (semi0, semi1)
    osems = (semo0, semo1)

    def chunk_row(t):
        c = t * _NW + wid
        return pl.multiple_of(c * _RCH, 8)

    def issue_in(t, s):
        r = chunk_row(t)
        hx = pltpu.async_copy(x_hbm.at[pl.ds(r, _RCH), :], xbufs[s], isems[s])
        hk = pltpu.async_copy(k_hbm.at[pl.ds(r, _RCH), :], kbufs[s], isems[s])
        return (hx, hk)

    def issue_out(t, s):
        r = chunk_row(t)
        return pltpu.async_copy(cbufs[s], out_hbm.at[pl.ds(r, _RCH), :],
                                osems[s])

    def compute(s):
        xb = xbufs[s]
        kb = kbufs[s]
        cb = cbufs[s]

        def group_b(g, _):
            off = pl.multiple_of(g * 16, 16)
            i0g = tiv[pl.ds(off, 16)]
            i1g = tiv[pl.ds(off + _F, 16)]
            w0g = twv[pl.ds(off, 16)]
            w1g = twv[pl.ds(off + _F, 16)]

            def row_b(r, _):
                rx = jnp.full((16,), 0, jnp.int32) + r
                g0 = plsc.load_gather(xb, [rx, i0g])
                g1 = plsc.load_gather(xb, [rx, i1g])
                wt = g0 * w0g + g1 * w1g
                kf = kb[r, pl.ds(off, 16)]
                xv = xb[r, pl.ds(off, 16)]
                cb[r, pl.ds(off, 16)] = jnp.where(
                    kf != zero, xv * scale_keep, wt)
                return 0

            lax.fori_loop(0, _RCH, row_b, 0, unroll=4)
            return 0

        lax.fori_loop(0, 8, group_b, 0, unroll=False)

    in_h = [None, None]
    out_h = [None, None]
    in_h[0] = issue_in(0, 0)
    for t in range(_NCHW):
        s = t % 2
        if t + 1 < _NCHW:
            in_h[1 - s] = issue_in(t + 1, 1 - s)
        hx, hk = in_h[s]
        hx.wait()
        hk.wait()
        if out_h[s] is not None:
            out_h[s].wait()
        compute(s)
        out_h[s] = issue_out(t, s)
    out_h[0].wait()
    out_h[1].wait()


def _tc_tail_body(x_ref, k_ref, st_ref, o_ref):
    wt = jnp.dot(x_ref[...], st_ref[...],
                 precision=lax.Precision.HIGHEST,
                 preferred_element_type=jnp.float32)
    o_ref[...] = jnp.where(k_ref[...],
                           x_ref[...] * jnp.float32(1.0 / (1.0 - _P)),
                           wt * jnp.float32(1.0 / (_P + 1e-12)))


def _tc_tail(x, keep, st):
    base = _BSC // _TROWS
    return pl.pallas_call(
        _tc_tail_body,
        out_shape=jax.ShapeDtypeStruct((_B, _F), jnp.float32),
        grid=((_B - _BSC) // _TROWS,),
        in_specs=[pl.BlockSpec((_TROWS, _F), lambda i: (base + i, 0)),
                  pl.BlockSpec((_TROWS, _F), lambda i: (base + i, 0)),
                  pl.BlockSpec((_F, _F), lambda i: (0, 0))],
        out_specs=pl.BlockSpec((_TROWS, _F), lambda i: (base + i, 0)),
    )(x, keep, st)


def kernel(x, m_w, m_idx, keep):
    kf32 = keep[:_BSC].astype(jnp.float32)

    midx = m_idx.astype(jnp.int32)
    inv_p = jnp.float32(1.0 / (_P + 1e-12))
    ti = jnp.concatenate([midx[:, 0], midx[:, 1]])
    tw = jnp.concatenate([m_w[:, 0] * inv_p, m_w[:, 1] * inv_p])

    rows = jnp.arange(_F, dtype=jnp.int32)[:, None]
    st = (jnp.where(midx[None, :, 0] == rows, m_w[None, :, 0], 0.0)
          + jnp.where(midx[None, :, 1] == rows, m_w[None, :, 1], 0.0))

    mesh = plsc.VectorSubcoreMesh(core_axis_name="c", subcore_axis_name="s")
    out = pl.kernel(
        _sc_body,
        out_type=jax.ShapeDtypeStruct((_BSC, _F), jnp.float32),
        mesh=mesh,
        compiler_params=pltpu.CompilerParams(needs_layout_passes=False),
        scratch_types=[
            pltpu.VMEM((_RCH, _F), jnp.float32),
            pltpu.VMEM((_RCH, _F), jnp.float32),
            pltpu.VMEM((_RCH, _F), jnp.float32),
            pltpu.VMEM((_RCH, _F), jnp.float32),
            pltpu.VMEM((_RCH, _F), jnp.float32),
            pltpu.VMEM((_RCH, _F), jnp.float32),
            pltpu.VMEM((_F * 2,), jnp.int32),
            pltpu.VMEM((_F * 2,), jnp.float32),
            pltpu.SemaphoreType.DMA,
            pltpu.SemaphoreType.DMA,
            pltpu.SemaphoreType.DMA,
            pltpu.SemaphoreType.DMA,
        ],
    )(x, kf32, ti, tw)
    tail = _tc_tail(x, keep, st)
    return lax.dynamic_update_slice(tail, out, (0, 0))


# speedup vs baseline: 2.4510x; 1.0937x over previous
import jax
import jax.numpy as jnp
from jax import lax
from jax.experimental import pallas as pl
from jax.experimental.pallas import tpu as pltpu
from jax.experimental.pallas import tpu_sc as plsc

_P = 0.2
_B = 100000
_F = 128
_NC = 2
_NS = 16
_NW = _NC * _NS
_RCH = 120
_NCHW = 10
_BSC = _NW * _NCHW * _RCH
_TROWS = 800


def _sc_body(x_hbm, k_hbm, ti_hbm, tw_hbm, out_hbm,
             xb0, xb1, kb0, kb1, cb0, cb1, tiv, twv,
             semi0, semi1, semo0, semo1):
    wid = lax.axis_index("s") * _NC + lax.axis_index("c")

    pltpu.sync_copy(ti_hbm, tiv)
    pltpu.sync_copy(tw_hbm, twv)

    zero = jnp.float32(0.0)
    scale_keep = jnp.float32(1.0 / (1.0 - _P))

    xbufs = (xb0, xb1)
    kbufs = (kb0, kb1)
    cbufs = (cb0, cb1)
    isems = (semi0, semi1)
    osems = (semo0, semo1)

    def chunk_row(t):
        c = t * _NW + wid
        return pl.multiple_of(c * _RCH, 8)

    def issue_in(t, s):
        r = chunk_row(t)
        hx = pltpu.async_copy(x_hbm.at[pl.ds(r, _RCH), :], xbufs[s], isems[s])
        hk = pltpu.async_copy(k_hbm.at[pl.ds(r, _RCH), :], kbufs[s], isems[s])
        return (hx, hk)

    def issue_out(t, s):
        r = chunk_row(t)
        return pltpu.async_copy(cbufs[s], out_hbm.at[pl.ds(r, _RCH), :],
                                osems[s])

    def compute(s):
        xb = xbufs[s]
        kb = kbufs[s]
        cb = cbufs[s]

        def group_b(g, _):
            off = pl.multiple_of(g * 16, 16)
            i0g = tiv[pl.ds(off, 16)]
            i1g = tiv[pl.ds(off + _F, 16)]
            w0g = twv[pl.ds(off, 16)]
            w1g = twv[pl.ds(off + _F, 16)]

            def row_b(r, _):
                rx = jnp.full((16,), 0, jnp.int32) + r
                g0 = plsc.load_gather(xb, [rx, i0g])
                g1 = plsc.load_gather(xb, [rx, i1g])
                wt = g0 * w0g + g1 * w1g
                kf = kb[r, pl.ds(off, 16)]
                xv = xb[r, pl.ds(off, 16)]
                cb[r, pl.ds(off, 16)] = jnp.where(
                    kf != zero, xv * scale_keep, wt)
                return 0

            lax.fori_loop(0, _RCH, row_b, 0, unroll=4)
            return 0

        lax.fori_loop(0, 8, group_b, 0, unroll=False)

    in_h = [None, None]
    out_h = [None, None]
    in_h[0] = issue_in(0, 0)
    for t in range(_NCHW):
        s = t % 2
        if t + 1 < _NCHW:
            in_h[1 - s] = issue_in(t + 1, 1 - s)
        hx, hk = in_h[s]
        hx.wait()
        hk.wait()
        if out_h[s] is not None:
            out_h[s].wait()
        compute(s)
        out_h[s] = issue_out(t, s)
    out_h[0].wait()
    out_h[1].wait()


def _tc_tail_body(x_ref, k_ref, st_ref, o_ref):
    wt = jnp.dot(x_ref[...], st_ref[...],
                 precision=lax.Precision.HIGHEST,
                 preferred_element_type=jnp.float32)
    o_ref[...] = jnp.where(k_ref[...] != 0,
                           x_ref[...] * jnp.float32(1.0 / (1.0 - _P)),
                           wt * jnp.float32(1.0 / (_P + 1e-12)))


def _tc_tail(x, keep, st):
    base = _BSC // _TROWS
    return pl.pallas_call(
        _tc_tail_body,
        out_shape=jax.ShapeDtypeStruct((_B, _F), jnp.float32),
        grid=((_B - _BSC) // _TROWS,),
        in_specs=[pl.BlockSpec((_TROWS, _F), lambda i: (base + i, 0)),
                  pl.BlockSpec((_TROWS, _F), lambda i: (i, 0)),
                  pl.BlockSpec((_F, _F), lambda i: (0, 0))],
        out_specs=pl.BlockSpec((_TROWS, _F), lambda i: (base + i, 0)),
    )(x, keep[_BSC:].astype(jnp.int8), st)


def kernel(x, m_w, m_idx, keep):
    kf32 = keep[:_BSC].astype(jnp.float32)

    midx = m_idx.astype(jnp.int32)
    inv_p = jnp.float32(1.0 / (_P + 1e-12))
    ti = jnp.concatenate([midx[:, 0], midx[:, 1]])
    tw = jnp.concatenate([m_w[:, 0] * inv_p, m_w[:, 1] * inv_p])

    rows = jnp.arange(_F, dtype=jnp.int32)[:, None]
    st = (jnp.where(midx[None, :, 0] == rows, m_w[None, :, 0], 0.0)
          + jnp.where(midx[None, :, 1] == rows, m_w[None, :, 1], 0.0))

    mesh = plsc.VectorSubcoreMesh(core_axis_name="c", subcore_axis_name="s")
    out = pl.kernel(
        _sc_body,
        out_type=jax.ShapeDtypeStruct((_BSC, _F), jnp.float32),
        mesh=mesh,
        compiler_params=pltpu.CompilerParams(needs_layout_passes=False),
        scratch_types=[
            pltpu.VMEM((_RCH, _F), jnp.float32),
            pltpu.VMEM((_RCH, _F), jnp.float32),
            pltpu.VMEM((_RCH, _F), jnp.float32),
            pltpu.VMEM((_RCH, _F), jnp.float32),
            pltpu.VMEM((_RCH, _F), jnp.float32),
            pltpu.VMEM((_RCH, _F), jnp.float32),
            pltpu.VMEM((_F * 2,), jnp.int32),
            pltpu.VMEM((_F * 2,), jnp.float32),
            pltpu.SemaphoreType.DMA,
            pltpu.SemaphoreType.DMA,
            pltpu.SemaphoreType.DMA,
            pltpu.SemaphoreType.DMA,
        ],
    )(x, kf32, ti, tw)
    tail = _tc_tail(x, keep, st)
    return lax.dynamic_update_slice(tail, out, (0, 0))


# speedup vs baseline: 2.5263x; 1.0307x over previous
import jax
import jax.numpy as jnp
from jax import lax
from jax.experimental import pallas as pl
from jax.experimental.pallas import tpu as pltpu
from jax.experimental.pallas import tpu_sc as plsc

_P = 0.2
_B = 100000
_F = 128
_NC = 2
_NS = 16
_NW = _NC * _NS
_RCH = 40
_NCHW = 25
_BSC = _NW * _NCHW * _RCH
_TROWS = 800


def _sc_body(x_hbm, k_hbm, ti_hbm, tw_hbm, out_hbm,
             xb0, xb1, kb0, kb1, cb0, cb1, tiv, twv,
             semi0, semi1, semo0, semo1):
    wid = lax.axis_index("s") * _NC + lax.axis_index("c")

    pltpu.sync_copy(ti_hbm, tiv)
    pltpu.sync_copy(tw_hbm, twv)

    zero = jnp.float32(0.0)
    scale_keep = jnp.float32(1.0 / (1.0 - _P))

    xbufs = (xb0, xb1)
    kbufs = (kb0, kb1)
    cbufs = (cb0, cb1)
    isems = (semi0, semi1)
    osems = (semo0, semo1)

    def chunk_row(t):
        c = t * _NW + wid
        return pl.multiple_of(c * _RCH, 8)

    def issue_in(t, s):
        r = chunk_row(t)
        hx = pltpu.async_copy(x_hbm.at[pl.ds(r, _RCH), :], xbufs[s], isems[s])
        hk = pltpu.async_copy(k_hbm.at[pl.ds(r, _RCH), :], kbufs[s], isems[s])
        return (hx, hk)

    def issue_out(t, s):
        r = chunk_row(t)
        return pltpu.async_copy(cbufs[s], out_hbm.at[pl.ds(r, _RCH), :],
                                osems[s])

    def compute(s):
        xb = xbufs[s]
        kb = kbufs[s]
        cb = cbufs[s]

        def group_b(g, _):
            off = pl.multiple_of(g * 16, 16)
            i0g = tiv[pl.ds(off, 16)]
            i1g = tiv[pl.ds(off + _F, 16)]
            w0g = twv[pl.ds(off, 16)]
            w1g = twv[pl.ds(off + _F, 16)]

            def row_b(r, _):
                rx = jnp.full((16,), 0, jnp.int32) + r
                g0 = plsc.load_gather(xb, [rx, i0g])
                g1 = plsc.load_gather(xb, [rx, i1g])
                wt = g0 * w0g + g1 * w1g
                kf = kb[r, pl.ds(off, 16)]
                xv = xb[r, pl.ds(off, 16)]
                cb[r, pl.ds(off, 16)] = jnp.where(
                    kf != zero, xv * scale_keep, wt)
                return 0

            lax.fori_loop(0, _RCH, row_b, 0, unroll=4)
            return 0

        lax.fori_loop(0, 8, group_b, 0, unroll=False)

    in_h = [None, None]
    out_h = [None, None]
    in_h[0] = issue_in(0, 0)
    for t in range(_NCHW):
        s = t % 2
        if t + 1 < _NCHW:
            in_h[1 - s] = issue_in(t + 1, 1 - s)
        hx, hk = in_h[s]
        hx.wait()
        hk.wait()
        if out_h[s] is not None:
            out_h[s].wait()
        compute(s)
        out_h[s] = issue_out(t, s)
    out_h[0].wait()
    out_h[1].wait()


def _tc_tail_body(x_ref, k_ref, st_ref, o_ref):
    wt = jnp.dot(x_ref[...], st_ref[...],
                 precision=lax.Precision.HIGHEST,
                 preferred_element_type=jnp.float32)
    o_ref[...] = jnp.where(k_ref[...] != 0,
                           x_ref[...] * jnp.float32(1.0 / (1.0 - _P)),
                           wt * jnp.float32(1.0 / (_P + 1e-12)))


def _tc_tail(x, keep, st):
    base = _BSC // _TROWS
    return pl.pallas_call(
        _tc_tail_body,
        out_shape=jax.ShapeDtypeStruct((_B, _F), jnp.float32),
        grid=((_B - _BSC) // _TROWS,),
        in_specs=[pl.BlockSpec((_TROWS, _F), lambda i: (base + i, 0)),
                  pl.BlockSpec((_TROWS, _F), lambda i: (i, 0)),
                  pl.BlockSpec((_F, _F), lambda i: (0, 0))],
        out_specs=pl.BlockSpec((_TROWS, _F), lambda i: (base + i, 0)),
    )(x, keep[_BSC:].astype(jnp.int8), st)


def kernel(x, m_w, m_idx, keep):
    kf32 = keep[:_BSC].astype(jnp.float32)

    midx = m_idx.astype(jnp.int32)
    inv_p = jnp.float32(1.0 / (_P + 1e-12))
    ti = jnp.concatenate([midx[:, 0], midx[:, 1]])
    tw = jnp.concatenate([m_w[:, 0] * inv_p, m_w[:, 1] * inv_p])

    rows = jnp.arange(_F, dtype=jnp.int32)[:, None]
    st = (jnp.where(midx[None, :, 0] == rows, m_w[None, :, 0], 0.0)
          + jnp.where(midx[None, :, 1] == rows, m_w[None, :, 1], 0.0))

    mesh = plsc.VectorSubcoreMesh(core_axis_name="c", subcore_axis_name="s")
    out = pl.kernel(
        _sc_body,
        out_type=jax.ShapeDtypeStruct((_BSC, _F), jnp.float32),
        mesh=mesh,
        compiler_params=pltpu.CompilerParams(needs_layout_passes=False),
        scratch_types=[
            pltpu.VMEM((_RCH, _F), jnp.float32),
            pltpu.VMEM((_RCH, _F), jnp.float32),
            pltpu.VMEM((_RCH, _F), jnp.float32),
            pltpu.VMEM((_RCH, _F), jnp.float32),
            pltpu.VMEM((_RCH, _F), jnp.float32),
            pltpu.VMEM((_RCH, _F), jnp.float32),
            pltpu.VMEM((_F * 2,), jnp.int32),
            pltpu.VMEM((_F * 2,), jnp.float32),
            pltpu.SemaphoreType.DMA,
            pltpu.SemaphoreType.DMA,
            pltpu.SemaphoreType.DMA,
            pltpu.SemaphoreType.DMA,
        ],
    )(x, kf32, ti, tw)
    tail = _tc_tail(x, keep, st)
    return lax.dynamic_update_slice(tail, out, (0, 0))
